# baseline (speedup 1.0000x reference)
import dataclasses
import functools

import jax
import jax.numpy as jnp
import numpy as np
from jax import lax
from jax.experimental import pallas as pl
from jax.experimental.pallas import tpu as pltpu
from jax.experimental.pallas import tpu_sc as plsc

N = 10000
D = 256
K = 16
NCLS = 3

NW = 32
NP = 10240
W_PER = NP // NW
IDX_PER = W_PER * K
CH_N = 8
CH_ROWS = CH_N * K
NCH = W_PER // CH_N

_INV_DEG = 1.0 / 17.0


@functools.cache
def _vmesh():
    return plsc.VectorSubcoreMesh(core_axis_name="c", subcore_axis_name="s")


def _sc_compiler_params():
    cp = pltpu.CompilerParams()
    if "needs_layout_passes" in pltpu.CompilerParams.__dataclass_fields__:
        cp = dataclasses.replace(cp, needs_layout_passes=False)
    return cp


RB = 256


NG = NP // 128
TL = 4
BIGI = 2**30


def _knn_body(crow_ref, cxc_ref, cyc_ref, out_ref, d_ref):
    i = pl.program_id(0)
    inf = jnp.float32(jnp.inf)
    cx = crow_ref[:, 0:1].reshape(RB, 1, 1)
    cy = crow_ref[:, 1:2].reshape(RB, 1, 1)
    axc = cxc_ref[...][None, :, :]
    ayc = cyc_ref[...][None, :, :]
    dx = cx - axc
    dy = cy - ayc
    d = dx * dx + dy * dy
    row3 = (i * RB + lax.broadcasted_iota(jnp.int32, (RB, 1, 1), 0))
    g3 = lax.broadcasted_iota(jnp.int32, (RB, NG, 128), 1)
    l3 = lax.broadcasted_iota(jnp.int32, (RB, NG, 128), 2)
    col3 = g3 * 128 + l3
    d = jnp.where(col3 == row3, inf, d)
    d_ref[...] = d

    lane2 = lax.broadcasted_iota(jnp.int32, (RB, 128), 1)
    vals = []
    cols = []
    for _ in range(TL):
        d = d_ref[...]
        m = jnp.min(d, axis=1)
        g = jnp.min(jnp.where(d == m[:, None, :], g3, BIGI), axis=1)
        vals.append(m)
        cols.append(g * 128 + lane2)
        d_ref[...] = jnp.where(g3 == g[:, None, :], inf, d)
    cand = jnp.concatenate(vals, axis=1)
    ccol = jnp.concatenate(cols, axis=1)

    v16 = None
    for k in range(K):
        m = jnp.min(cand, axis=1, keepdims=True)
        idx = jnp.min(jnp.where(cand == m, ccol, BIGI), axis=1, keepdims=True)
        out_ref[:, k : k + 1] = idx
        cand = jnp.where(ccol == idx, inf, cand)
        v16 = m

    ok = jnp.min(vals[TL - 1], axis=1, keepdims=True) > v16
    all_ok = jnp.min(jnp.where(ok, 1, 0)) == 1

    @pl.when(jnp.logical_not(all_ok))
    def _fallback():
        d = dx * dx + dy * dy
        d_ref[...] = jnp.where(col3 == row3, inf, d)
        for k in range(K):
            d = d_ref[...]
            m = jnp.min(jnp.min(d, axis=2), axis=1).reshape(RB, 1, 1)
            idx = jnp.min(
                jnp.min(jnp.where(d == m, col3, BIGI), axis=2), axis=1
            ).reshape(RB, 1)
            out_ref[:, k : k + 1] = idx
            d_ref[...] = jnp.where(col3 == idx[:, :, None], inf, d)


def _knn(centers_rows, cxc, cyc):
    return pl.pallas_call(
        _knn_body,
        grid=(NP // RB,),
        in_specs=[
            pl.BlockSpec((RB, 2), lambda i: (i, 0)),
            pl.BlockSpec((NG, 128), lambda i: (0, 0)),
            pl.BlockSpec((NG, 128), lambda i: (0, 0)),
        ],
        out_specs=pl.BlockSpec((RB, K), lambda i: (i, 0)),
        out_shape=jax.ShapeDtypeStruct((NP, K), jnp.int32),
        scratch_shapes=[pltpu.VMEM((RB, NG, 128), jnp.float32)],
    )(centers_rows, cxc, cyc)


MB = 512


def _mm1_body(x_ref, w_ref, o_ref):
    o_ref[...] = jnp.dot(x_ref[...], w_ref[...],
                         preferred_element_type=jnp.float32)


def _mm1(x, w):
    return pl.pallas_call(
        _mm1_body,
        grid=(NP // MB,),
        in_specs=[
            pl.BlockSpec((MB, D), lambda i: (i, 0)),
            pl.BlockSpec((D, D), lambda i: (0, 0)),
        ],
        out_specs=pl.BlockSpec((MB, D), lambda i: (i, 0)),
        out_shape=jax.ShapeDtypeStruct((NP, D), jnp.float32),
    )(x, w)


def _fuse1_body(s_ref, xw_ref, b_ref, w_ref, o_ref):
    x = jax.nn.relu((s_ref[...] + xw_ref[...]) * _INV_DEG + b_ref[0:1, :])
    o_ref[...] = jnp.dot(x, w_ref[...], preferred_element_type=jnp.float32)


def _fuse1(s, xw, b_pad, w):
    return pl.pallas_call(
        _fuse1_body,
        grid=(NP // MB,),
        in_specs=[
            pl.BlockSpec((MB, D), lambda i: (i, 0)),
            pl.BlockSpec((MB, D), lambda i: (i, 0)),
            pl.BlockSpec((8, D), lambda i: (0, 0)),
            pl.BlockSpec((D, D), lambda i: (0, 0)),
        ],
        out_specs=pl.BlockSpec((MB, D), lambda i: (i, 0)),
        out_shape=jax.ShapeDtypeStruct((NP, D), jnp.float32),
    )(s, xw, b_pad, w)


def _fuse2_body(s_ref, xw_ref, b_ref, w_ref, t_ref, bmat_ref):
    x = jax.nn.relu((s_ref[...] + xw_ref[...]) * _INV_DEG + b_ref[0:1, :])
    ab = jnp.dot(x, w_ref[...], preferred_element_type=jnp.float32)
    t_ref[...] = ab[:, 0:D]
    bmat_ref[...] = ab[:, D : 2 * D]


def _fuse2(s, xw, b_pad, wcat):
    return pl.pallas_call(
        _fuse2_body,
        grid=(NP // MB,),
        in_specs=[
            pl.BlockSpec((MB, D), lambda i: (i, 0)),
            pl.BlockSpec((MB, D), lambda i: (i, 0)),
            pl.BlockSpec((8, D), lambda i: (0, 0)),
            pl.BlockSpec((D, 2 * D), lambda i: (0, 0)),
        ],
        out_specs=[
            pl.BlockSpec((MB, D), lambda i: (i, 0)),
            pl.BlockSpec((MB, D), lambda i: (i, 0)),
        ],
        out_shape=[
            jax.ShapeDtypeStruct((NP, D), jnp.float32),
            jax.ShapeDtypeStruct((NP, D), jnp.float32),
        ],
    )(s, xw, b_pad, wcat)


GS_N = 4
GS_ROWS = GS_N * K
GS_NCH = W_PER // GS_N


def _gsum_sc(table, idx_flat):
    @functools.partial(
        pl.kernel,
        out_type=jax.ShapeDtypeStruct((NP, D), jnp.float32),
        mesh=_vmesh(),
        scratch_types=[
            pltpu.VMEM((IDX_PER,), jnp.int32),
            pltpu.VMEM((2, GS_ROWS, D), jnp.float32),
            pltpu.VMEM((W_PER, D), jnp.float32),
            pltpu.SemaphoreType.DMA,
            pltpu.SemaphoreType.DMA,
        ],
    )
    def body(table_hbm, idx_hbm, out_hbm, idx_v, rows_v, acc_v, sem0, sem1):
        wid = lax.axis_index("s") * 2 + lax.axis_index("c")
        pltpu.sync_copy(idx_hbm.at[pl.ds(wid * IDX_PER, IDX_PER)], idx_v)

        def gcopy(ch, buf, sem):
            return pltpu.make_async_copy(
                table_hbm.at[idx_v.at[pl.ds(ch * GS_ROWS, GS_ROWS)]],
                rows_v.at[buf],
                sem,
            )

        def sum_chunk(buf, ch):
            @pl.loop(0, GS_N)
            def _node(n):
                @pl.loop(0, D, step=16)
                def _col(j):
                    a = rows_v[buf, n * K, pl.ds(j, 16)]
                    for r in range(1, K):
                        a = a + rows_v[buf, n * K + r, pl.ds(j, 16)]
                    acc_v[ch * GS_N + n, pl.ds(j, 16)] = a

        gcopy(0, 0, sem0).start()

        @pl.loop(0, GS_NCH, step=2)
        def _chunk(ch):
            gcopy(ch + 1, 1, sem1).start()
            gcopy(ch, 0, sem0).wait()
            sum_chunk(0, ch)

            @pl.when(ch + 2 < GS_NCH)
            def _():
                gcopy(ch + 2, 0, sem0).start()

            gcopy(ch + 1, 1, sem1).wait()
            sum_chunk(1, ch + 1)

        pltpu.sync_copy(acc_v, out_hbm.at[pl.ds(wid * W_PER, W_PER)])

    return body(table, idx_flat)


def _gather_sc(table, bb4, idx_flat):
    @functools.partial(
        pl.kernel,
        out_type=[
            jax.ShapeDtypeStruct((NP * K, D), jnp.float32),
            jax.ShapeDtypeStruct((NP * K * 4,), jnp.float32),
        ],
        mesh=_vmesh(),
        scratch_types=[
            pltpu.VMEM((IDX_PER,), jnp.int32),
            pltpu.VMEM((2, CH_ROWS, D), jnp.float32),
            pltpu.VMEM((NP * 4,), jnp.float32),
            pltpu.VMEM((2 * CH_ROWS * 4,), jnp.float32),
            pltpu.SemaphoreType.DMA,
            pltpu.SemaphoreType.DMA,
            pltpu.SemaphoreType.DMA,
            pltpu.SemaphoreType.DMA,
            pltpu.SemaphoreType.DMA,
            pltpu.SemaphoreType.DMA,
        ],
        compiler_params=_sc_compiler_params(),
    )
    def body(table_hbm, bb_hbm, idx_hbm, out_hbm, bbo_hbm, idx_v, rows_v,
             bbt_v, bbuf_v, in0, in1, out0, out1, sb0, sb1):
        wid = lax.axis_index("s") * 2 + lax.axis_index("c")
        pltpu.sync_copy(idx_hbm.at[pl.ds(wid * IDX_PER, IDX_PER)], idx_v)
        pltpu.sync_copy(bb_hbm, bbt_v)
        lane = lax.iota(jnp.int32, 16)

        def gcopy(ch, buf, sem):
            return pltpu.make_async_copy(
                table_hbm.at[idx_v.at[pl.ds(ch * CH_ROWS, CH_ROWS)]],
                rows_v.at[buf],
                sem,
            )

        def rows_out(ch, buf, sem):
            return pltpu.make_async_copy(
                rows_v.at[buf],
                out_hbm.at[pl.ds(wid * IDX_PER + ch * CH_ROWS, CH_ROWS)],
                sem,
            )

        def bb_out(ch, buf, sem):
            return pltpu.make_async_copy(
                bbuf_v.at[pl.ds(buf * CH_ROWS * 4, CH_ROWS * 4)],
                bbo_hbm.at[pl.ds((wid * IDX_PER + ch * CH_ROWS) * 4,
                                 CH_ROWS * 4)],
                sem,
            )

        def bb_chunk(ch, buf):
            for g in range(CH_ROWS // 16):
                idx16 = idx_v[pl.ds(ch * CH_ROWS + g * 16, 16)]
                for c in range(4):
                    cvec = jnp.full((16,), c, jnp.int32)
                    vals = plsc.load_gather(bbt_v, [idx16 * 4 + cvec])
                    plsc.store_scatter(
                        bbuf_v,
                        [buf * (CH_ROWS * 4) + (g * 16 + lane) * 4 + cvec],
                        vals,
                    )

        gcopy(0, 0, in0).start()

        @pl.loop(0, NCH, step=2)
        def _chunk(ch):
            @pl.when(ch > 0)
            def _():
                rows_out(ch, 1, out1).wait()
            gcopy(ch + 1, 1, in1).start()
            gcopy(ch, 0, in0).wait()

            @pl.when(ch > 0)
            def _():
                bb_out(ch, 0, sb0).wait()
            bb_chunk(ch, 0)
            rows_out(ch, 0, out0).start()
            bb_out(ch, 0, sb0).start()

            @pl.when(ch + 2 < NCH)
            def _():
                rows_out(ch, 0, out0).wait()
                gcopy(ch + 2, 0, in0).start()
            gcopy(ch + 1, 1, in1).wait()

            @pl.when(ch > 0)
            def _():
                bb_out(ch, 1, sb1).wait()
            bb_chunk(ch + 1, 1)
            rows_out(ch + 1, 1, out1).start()
            bb_out(ch + 1, 1, sb1).start()

        rows_out(0, 0, out0).wait()
        rows_out(0, 1, out1).wait()
        bb_out(0, 0, sb0).wait()
        bb_out(0, 1, sb1).wait()

    return body(table, bb4, idx_flat)


FB = 128
FE = FB * K


def _final_body(p_ref, bbsrc_ref, bmat_ref, bb_ref, wfin_ref, blin_ref,
                bfin_ref, probs_ref, bbox_ref):
    a_src = p_ref[...]
    brep = jnp.broadcast_to(
        bmat_ref[...][:, None, :], (FB, K, D)
    ).reshape(FE, D)
    h = jax.nn.relu(a_src + brep + blin_ref[0:1, :])
    logits = jnp.dot(h, wfin_ref[...], preferred_element_type=jnp.float32)
    logits = logits + bfin_ref[0:1, :]
    mask = lax.broadcasted_iota(jnp.int32, (FE, 128), 1) < NCLS
    neg = jnp.float32(-jnp.inf)
    m = jnp.max(jnp.where(mask, logits, neg), axis=1, keepdims=True)
    sh = logits - m
    ssum = jnp.sum(jnp.where(mask, jnp.exp(sh), 0.0), axis=1, keepdims=True)
    ls = sh - jnp.log(ssum)
    probs_ref[...] = ls[:, 0:NCLS]
    bb_src = bbsrc_ref[...]
    bb_dst = jnp.broadcast_to(
        bb_ref[...][:, None, :], (FB, K, 4)
    ).reshape(FE, 4)
    bbox_ref[...] = jnp.concatenate([bb_src, bb_dst], axis=1)


def _final(p, bbsrc, bmat, bbp, wfin_pad, blin_pad, bfin_pad):
    return pl.pallas_call(
        _final_body,
        grid=(NP // FB,),
        in_specs=[
            pl.BlockSpec((FE, D), lambda i: (i, 0)),
            pl.BlockSpec((FE, 4), lambda i: (i, 0)),
            pl.BlockSpec((FB, D), lambda i: (i, 0)),
            pl.BlockSpec((FB, 4), lambda i: (i, 0)),
            pl.BlockSpec((D, 128), lambda i: (0, 0)),
            pl.BlockSpec((8, D), lambda i: (0, 0)),
            pl.BlockSpec((8, 128), lambda i: (0, 0)),
        ],
        out_specs=[
            pl.BlockSpec((FE, NCLS), lambda i: (i, 0)),
            pl.BlockSpec((FE, 8), lambda i: (i, 0)),
        ],
        out_shape=[
            jax.ShapeDtypeStruct((NP * K, NCLS), jnp.float32),
            jax.ShapeDtypeStruct((NP * K, 8), jnp.float32),
        ],
    )(p, bbsrc, bmat, bbp, wfin_pad, blin_pad, bfin_pad)


def _pad_rows(x, rows, value=0.0):
    return jnp.pad(x, ((0, rows - x.shape[0]), (0, 0)), constant_values=value)


def kernel(hidden_state, pred_bboxes, W1, b1, W2, b2, W_lin1, b_lin1, W_fin,
           b_fin):
    hid = _pad_rows(hidden_state, NP)
    bbp = _pad_rows(pred_bboxes, NP, value=1e15)

    centers = (bbp[:, :2] + bbp[:, 2:4]) * 0.5
    cxc = centers[:, 0].reshape(NG, 128)
    cyc = centers[:, 1].reshape(NG, 128)

    nbr = _knn(centers, cxc, cyc)
    idx_flat = nbr.reshape(-1)

    b1p = jnp.pad(b1.reshape(1, D), ((0, 7), (0, 0)))
    b2p = jnp.pad(b2.reshape(1, D), ((0, 7), (0, 0)))
    blinp = jnp.pad(b_lin1.reshape(1, D), ((0, 7), (0, 0)))
    bfinp = jnp.pad(b_fin.reshape(1, NCLS), ((0, 7), (0, 128 - NCLS)))
    wcat = jnp.concatenate([W_lin1[:D, :], W_lin1[D:, :]], axis=1)
    wfinp = jnp.pad(W_fin, ((0, 0), (0, 128 - NCLS)))

    xw1 = _mm1(hid, W1)
    s1 = _gsum_sc(xw1, idx_flat)
    xw2 = _fuse1(s1, xw1, b1p, W2)
    s2 = _gsum_sc(xw2, idx_flat)
    t, bmat = _fuse2(s2, xw2, b2p, wcat)
    p, bbsrc_flat = _gather_sc(t, bbp.reshape(-1), idx_flat)
    bbsrc = bbsrc_flat.reshape(NP * K, 4)
    probs_full, bbox_full = _final(p, bbsrc, bmat, bbp, wfinp, blinp, bfinp)
    return probs_full[: N * K], bbox_full[: N * K]


# speedup vs baseline: 1.2525x; 1.2525x over previous
import dataclasses
import functools

import jax
import jax.numpy as jnp
import numpy as np
from jax import lax
from jax.experimental import pallas as pl
from jax.experimental.pallas import tpu as pltpu
from jax.experimental.pallas import tpu_sc as plsc

N = 10000
D = 256
K = 16
NCLS = 3

NW = 32
NP = 10240
W_PER = NP // NW
IDX_PER = W_PER * K
CH_N = 8
CH_ROWS = CH_N * K
NCH = W_PER // CH_N

_INV_DEG = 1.0 / 17.0


@functools.cache
def _vmesh():
    return plsc.VectorSubcoreMesh(core_axis_name="c", subcore_axis_name="s")


def _sc_compiler_params():
    cp = pltpu.CompilerParams()
    if "needs_layout_passes" in pltpu.CompilerParams.__dataclass_fields__:
        cp = dataclasses.replace(cp, needs_layout_passes=False)
    return cp


RB = 256


NG = NP // 128
TL = 4
BIGI = 2**30


def _knn_body(crow_ref, cxc_ref, cyc_ref, out_ref, d_ref):
    i = pl.program_id(0)
    inf = jnp.float32(jnp.inf)
    cx = crow_ref[:, 0:1].reshape(RB, 1, 1)
    cy = crow_ref[:, 1:2].reshape(RB, 1, 1)
    axc = cxc_ref[...][None, :, :]
    ayc = cyc_ref[...][None, :, :]
    dx = cx - axc
    dy = cy - ayc
    d = dx * dx + dy * dy
    row3 = (i * RB + lax.broadcasted_iota(jnp.int32, (RB, 1, 1), 0))
    g3 = lax.broadcasted_iota(jnp.int32, (RB, NG, 128), 1)
    l3 = lax.broadcasted_iota(jnp.int32, (RB, NG, 128), 2)
    col3 = g3 * 128 + l3
    d = jnp.where(col3 == row3, inf, d)
    d_ref[...] = d

    lane2 = lax.broadcasted_iota(jnp.int32, (RB, 128), 1)
    vals = []
    cols = []
    for _ in range(TL):
        d = d_ref[...]
        m = jnp.min(d, axis=1)
        g = jnp.min(jnp.where(d == m[:, None, :], g3, BIGI), axis=1)
        vals.append(m)
        cols.append(g * 128 + lane2)
        d_ref[...] = jnp.where(g3 == g[:, None, :], inf, d)
    guard = jnp.min(d_ref[...], axis=1)
    cand = jnp.concatenate(vals, axis=1)
    ccol = jnp.concatenate(cols, axis=1)

    v16 = None
    for k in range(K):
        m = jnp.min(cand, axis=1, keepdims=True)
        idx = jnp.min(jnp.where(cand == m, ccol, BIGI), axis=1, keepdims=True)
        out_ref[:, k : k + 1] = idx
        cand = jnp.where(ccol == idx, inf, cand)
        v16 = m

    ok = jnp.min(guard, axis=1, keepdims=True) > v16
    all_ok = jnp.min(jnp.where(ok, 1, 0)) == 1

    @pl.when(jnp.logical_not(all_ok))
    def _fallback():
        d = dx * dx + dy * dy
        d_ref[...] = jnp.where(col3 == row3, inf, d)
        for k in range(K):
            d = d_ref[...]
            m = jnp.min(jnp.min(d, axis=2), axis=1).reshape(RB, 1, 1)
            idx = jnp.min(
                jnp.min(jnp.where(d == m, col3, BIGI), axis=2), axis=1
            ).reshape(RB, 1)
            out_ref[:, k : k + 1] = idx
            d_ref[...] = jnp.where(col3 == idx[:, :, None], inf, d)


def _knn(centers_rows, cxc, cyc):
    return pl.pallas_call(
        _knn_body,
        grid=(NP // RB,),
        in_specs=[
            pl.BlockSpec((RB, 2), lambda i: (i, 0)),
            pl.BlockSpec((NG, 128), lambda i: (0, 0)),
            pl.BlockSpec((NG, 128), lambda i: (0, 0)),
        ],
        out_specs=pl.BlockSpec((RB, K), lambda i: (i, 0)),
        out_shape=jax.ShapeDtypeStruct((NP, K), jnp.int32),
        scratch_shapes=[pltpu.VMEM((RB, NG, 128), jnp.float32)],
    )(centers_rows, cxc, cyc)


MB = 512


def _mm1_body(x_ref, w_ref, o_ref):
    o_ref[...] = jnp.dot(x_ref[...], w_ref[...],
                         preferred_element_type=jnp.float32)


def _mm1(x, w):
    return pl.pallas_call(
        _mm1_body,
        grid=(NP // MB,),
        in_specs=[
            pl.BlockSpec((MB, D), lambda i: (i, 0)),
            pl.BlockSpec((D, D), lambda i: (0, 0)),
        ],
        out_specs=pl.BlockSpec((MB, D), lambda i: (i, 0)),
        out_shape=jax.ShapeDtypeStruct((NP, D), jnp.float32),
    )(x, w)


def _fuse1_body(s_ref, xw_ref, b_ref, w_ref, o_ref):
    x = jax.nn.relu((s_ref[...] + xw_ref[...]) * _INV_DEG + b_ref[0:1, :])
    o_ref[...] = jnp.dot(x, w_ref[...], preferred_element_type=jnp.float32)


def _fuse1(s, xw, b_pad, w):
    return pl.pallas_call(
        _fuse1_body,
        grid=(NP // MB,),
        in_specs=[
            pl.BlockSpec((MB, D), lambda i: (i, 0)),
            pl.BlockSpec((MB, D), lambda i: (i, 0)),
            pl.BlockSpec((8, D), lambda i: (0, 0)),
            pl.BlockSpec((D, D), lambda i: (0, 0)),
        ],
        out_specs=pl.BlockSpec((MB, D), lambda i: (i, 0)),
        out_shape=jax.ShapeDtypeStruct((NP, D), jnp.float32),
    )(s, xw, b_pad, w)


def _fuse2_body(s_ref, xw_ref, b_ref, w_ref, t_ref, bmat_ref):
    x = jax.nn.relu((s_ref[...] + xw_ref[...]) * _INV_DEG + b_ref[0:1, :])
    ab = jnp.dot(x, w_ref[...], preferred_element_type=jnp.float32)
    t_ref[...] = ab[:, 0:D]
    bmat_ref[...] = ab[:, D : 2 * D]


def _fuse2(s, xw, b_pad, wcat):
    return pl.pallas_call(
        _fuse2_body,
        grid=(NP // MB,),
        in_specs=[
            pl.BlockSpec((MB, D), lambda i: (i, 0)),
            pl.BlockSpec((MB, D), lambda i: (i, 0)),
            pl.BlockSpec((8, D), lambda i: (0, 0)),
            pl.BlockSpec((D, 2 * D), lambda i: (0, 0)),
        ],
        out_specs=[
            pl.BlockSpec((MB, D), lambda i: (i, 0)),
            pl.BlockSpec((MB, D), lambda i: (i, 0)),
        ],
        out_shape=[
            jax.ShapeDtypeStruct((NP, D), jnp.float32),
            jax.ShapeDtypeStruct((NP, D), jnp.float32),
        ],
    )(s, xw, b_pad, wcat)


GS_N = 4
GS_ROWS = GS_N * K
GS_NCH = W_PER // GS_N


def _gsum_sc(table, idx_flat):
    @functools.partial(
        pl.kernel,
        out_type=jax.ShapeDtypeStruct((NP, D), jnp.float32),
        mesh=_vmesh(),
        scratch_types=[
            pltpu.VMEM((IDX_PER,), jnp.int32),
            pltpu.VMEM((2, GS_ROWS, D), jnp.float32),
            pltpu.VMEM((W_PER, D), jnp.float32),
            pltpu.SemaphoreType.DMA,
            pltpu.SemaphoreType.DMA,
        ],
    )
    def body(table_hbm, idx_hbm, out_hbm, idx_v, rows_v, acc_v, sem0, sem1):
        wid = lax.axis_index("s") * 2 + lax.axis_index("c")
        pltpu.sync_copy(idx_hbm.at[pl.ds(wid * IDX_PER, IDX_PER)], idx_v)

        def gcopy(ch, buf, sem):
            return pltpu.make_async_copy(
                table_hbm.at[idx_v.at[pl.ds(ch * GS_ROWS, GS_ROWS)]],
                rows_v.at[buf],
                sem,
            )

        def sum_chunk(buf, ch):
            @pl.loop(0, GS_N)
            def _node(n):
                @pl.loop(0, D, step=16)
                def _col(j):
                    a = rows_v[buf, n * K, pl.ds(j, 16)]
                    for r in range(1, K):
                        a = a + rows_v[buf, n * K + r, pl.ds(j, 16)]
                    acc_v[ch * GS_N + n, pl.ds(j, 16)] = a

        gcopy(0, 0, sem0).start()

        @pl.loop(0, GS_NCH, step=2)
        def _chunk(ch):
            gcopy(ch + 1, 1, sem1).start()
            gcopy(ch, 0, sem0).wait()
            sum_chunk(0, ch)

            @pl.when(ch + 2 < GS_NCH)
            def _():
                gcopy(ch + 2, 0, sem0).start()

            gcopy(ch + 1, 1, sem1).wait()
            sum_chunk(1, ch + 1)

        pltpu.sync_copy(acc_v, out_hbm.at[pl.ds(wid * W_PER, W_PER)])

    return body(table, idx_flat)


def _gather_sc(table, bb4, idx_flat):
    @functools.partial(
        pl.kernel,
        out_type=[
            jax.ShapeDtypeStruct((NP * K, D), jnp.float32),
            jax.ShapeDtypeStruct((NP * K * 4,), jnp.float32),
        ],
        mesh=_vmesh(),
        scratch_types=[
            pltpu.VMEM((IDX_PER,), jnp.int32),
            pltpu.VMEM((2, CH_ROWS, D), jnp.float32),
            pltpu.VMEM((NP * 4,), jnp.float32),
            pltpu.VMEM((2 * CH_ROWS * 4,), jnp.float32),
            pltpu.SemaphoreType.DMA,
            pltpu.SemaphoreType.DMA,
            pltpu.SemaphoreType.DMA,
            pltpu.SemaphoreType.DMA,
            pltpu.SemaphoreType.DMA,
            pltpu.SemaphoreType.DMA,
        ],
        compiler_params=_sc_compiler_params(),
    )
    def body(table_hbm, bb_hbm, idx_hbm, out_hbm, bbo_hbm, idx_v, rows_v,
             bbt_v, bbuf_v, in0, in1, out0, out1, sb0, sb1):
        wid = lax.axis_index("s") * 2 + lax.axis_index("c")
        pltpu.sync_copy(idx_hbm.at[pl.ds(wid * IDX_PER, IDX_PER)], idx_v)
        pltpu.sync_copy(bb_hbm, bbt_v)
        lane = lax.iota(jnp.int32, 16)

        def gcopy(ch, buf, sem):
            return pltpu.make_async_copy(
                table_hbm.at[idx_v.at[pl.ds(ch * CH_ROWS, CH_ROWS)]],
                rows_v.at[buf],
                sem,
            )

        def rows_out(ch, buf, sem):
            return pltpu.make_async_copy(
                rows_v.at[buf],
                out_hbm.at[pl.ds(wid * IDX_PER + ch * CH_ROWS, CH_ROWS)],
                sem,
            )

        def bb_out(ch, buf, sem):
            return pltpu.make_async_copy(
                bbuf_v.at[pl.ds(buf * CH_ROWS * 4, CH_ROWS * 4)],
                bbo_hbm.at[pl.ds((wid * IDX_PER + ch * CH_ROWS) * 4,
                                 CH_ROWS * 4)],
                sem,
            )

        def bb_chunk(ch, buf):
            for g in range(CH_ROWS // 16):
                idx16 = idx_v[pl.ds(ch * CH_ROWS + g * 16, 16)]
                for c in range(4):
                    cvec = jnp.full((16,), c, jnp.int32)
                    vals = plsc.load_gather(bbt_v, [idx16 * 4 + cvec])
                    plsc.store_scatter(
                        bbuf_v,
                        [buf * (CH_ROWS * 4) + (g * 16 + lane) * 4 + cvec],
                        vals,
                    )

        gcopy(0, 0, in0).start()

        @pl.loop(0, NCH, step=2)
        def _chunk(ch):
            @pl.when(ch > 0)
            def _():
                rows_out(ch, 1, out1).wait()
            gcopy(ch + 1, 1, in1).start()
            gcopy(ch, 0, in0).wait()

            @pl.when(ch > 0)
            def _():
                bb_out(ch, 0, sb0).wait()
            bb_chunk(ch, 0)
            rows_out(ch, 0, out0).start()
            bb_out(ch, 0, sb0).start()

            @pl.when(ch + 2 < NCH)
            def _():
                rows_out(ch, 0, out0).wait()
                gcopy(ch + 2, 0, in0).start()
            gcopy(ch + 1, 1, in1).wait()

            @pl.when(ch > 0)
            def _():
                bb_out(ch, 1, sb1).wait()
            bb_chunk(ch + 1, 1)
            rows_out(ch + 1, 1, out1).start()
            bb_out(ch + 1, 1, sb1).start()

        rows_out(0, 0, out0).wait()
        rows_out(0, 1, out1).wait()
        bb_out(0, 0, sb0).wait()
        bb_out(0, 1, sb1).wait()

    return body(table, bb4, idx_flat)


FB = 128
FE = FB * K


def _final_body(p_ref, bbsrc_ref, bmat_ref, bb_ref, wfin_ref, blin_ref,
                bfin_ref, probs_ref, bbox_ref):
    a_src = p_ref[...]
    brep = jnp.broadcast_to(
        bmat_ref[...][:, None, :], (FB, K, D)
    ).reshape(FE, D)
    h = jax.nn.relu(a_src + brep + blin_ref[0:1, :])
    logits = jnp.dot(h, wfin_ref[...], preferred_element_type=jnp.float32)
    logits = logits + bfin_ref[0:1, :]
    mask = lax.broadcasted_iota(jnp.int32, (FE, 128), 1) < NCLS
    neg = jnp.float32(-jnp.inf)
    m = jnp.max(jnp.where(mask, logits, neg), axis=1, keepdims=True)
    sh = logits - m
    ssum = jnp.sum(jnp.where(mask, jnp.exp(sh), 0.0), axis=1, keepdims=True)
    ls = sh - jnp.log(ssum)
    probs_ref[...] = ls[:, 0:NCLS]
    bb_src = bbsrc_ref[...]
    bb_dst = jnp.broadcast_to(
        bb_ref[...][:, None, :], (FB, K, 4)
    ).reshape(FE, 4)
    bbox_ref[...] = jnp.concatenate([bb_src, bb_dst], axis=1)


def _final(p, bbsrc, bmat, bbp, wfin_pad, blin_pad, bfin_pad):
    return pl.pallas_call(
        _final_body,
        grid=(NP // FB,),
        in_specs=[
            pl.BlockSpec((FE, D), lambda i: (i, 0)),
            pl.BlockSpec((FE, 4), lambda i: (i, 0)),
            pl.BlockSpec((FB, D), lambda i: (i, 0)),
            pl.BlockSpec((FB, 4), lambda i: (i, 0)),
            pl.BlockSpec((D, 128), lambda i: (0, 0)),
            pl.BlockSpec((8, D), lambda i: (0, 0)),
            pl.BlockSpec((8, 128), lambda i: (0, 0)),
        ],
        out_specs=[
            pl.BlockSpec((FE, NCLS), lambda i: (i, 0)),
            pl.BlockSpec((FE, 8), lambda i: (i, 0)),
        ],
        out_shape=[
            jax.ShapeDtypeStruct((NP * K, NCLS), jnp.float32),
            jax.ShapeDtypeStruct((NP * K, 8), jnp.float32),
        ],
    )(p, bbsrc, bmat, bbp, wfin_pad, blin_pad, bfin_pad)


def _pad_rows(x, rows, value=0.0):
    return jnp.pad(x, ((0, rows - x.shape[0]), (0, 0)), constant_values=value)


def kernel(hidden_state, pred_bboxes, W1, b1, W2, b2, W_lin1, b_lin1, W_fin,
           b_fin):
    hid = _pad_rows(hidden_state, NP)
    bbp = _pad_rows(pred_bboxes, NP, value=1e15)

    centers = (bbp[:, :2] + bbp[:, 2:4]) * 0.5
    cxc = centers[:, 0].reshape(NG, 128)
    cyc = centers[:, 1].reshape(NG, 128)

    nbr = _knn(centers, cxc, cyc)
    idx_flat = nbr.reshape(-1)

    b1p = jnp.pad(b1.reshape(1, D), ((0, 7), (0, 0)))
    b2p = jnp.pad(b2.reshape(1, D), ((0, 7), (0, 0)))
    blinp = jnp.pad(b_lin1.reshape(1, D), ((0, 7), (0, 0)))
    bfinp = jnp.pad(b_fin.reshape(1, NCLS), ((0, 7), (0, 128 - NCLS)))
    wcat = jnp.concatenate([W_lin1[:D, :], W_lin1[D:, :]], axis=1)
    wfinp = jnp.pad(W_fin, ((0, 0), (0, 128 - NCLS)))

    xw1 = _mm1(hid, W1)
    s1 = _gsum_sc(xw1, idx_flat)
    xw2 = _fuse1(s1, xw1, b1p, W2)
    s2 = _gsum_sc(xw2, idx_flat)
    t, bmat = _fuse2(s2, xw2, b2p, wcat)
    p, bbsrc_flat = _gather_sc(t, bbp.reshape(-1), idx_flat)
    bbsrc = bbsrc_flat.reshape(NP * K, 4)
    probs_full, bbox_full = _final(p, bbsrc, bmat, bbp, wfinp, blinp, bfinp)
    return probs_full[: N * K], bbox_full[: N * K]
